# Initial kernel scaffold; baseline (speedup 1.0000x reference)
import jax
import jax.numpy as jnp
from jax.experimental import pallas as pl


def kernel(y, Wxz, bxz, Whz, bhz, Wxr, bxr, Whr, bhr, Wxh, bxh, Whh, bhh, Wred, bred, Wm0, bm0, Wm1, bm1):
    raise NotImplementedError("write your pallas kernel here")


# speedup vs baseline: 1.9113x; 1.9113x over previous
import functools

import jax
import jax.numpy as jnp
from jax import lax
from jax.experimental import pallas as pl
from jax.experimental.pallas import tpu as pltpu

N = 256
FDIM = 128
HID = 16
T = 8


def _mm(a, b):
    return lax.dot_general(a, b, (((1,), (0,)), ((), ())),
                           preferred_element_type=jnp.float32)


def _mm_t(a, b):
    return lax.dot_general(a, b, (((0,), (0,)), ((), ())),
                           preferred_element_type=jnp.float32)


def _gru_kernel(y_ref, wx_ref, bx_ref, whzr_ref, bhzr_ref, whh_ref, bhh_ref,
                wred_ref, bred_ref, wm0_ref, bm0_ref, wm1_ref, bm1_ref,
                out_ref):
    row = lax.broadcasted_iota(jnp.int32, (N, N), 0)
    col = lax.broadcasted_iota(jnp.int32, (N, N), 1)
    offdiag = (row != col).astype(jnp.float32)

    wx0 = wx_ref[0]
    wx1 = wx_ref[1]
    wx2 = wx_ref[2]
    whzr0 = whzr_ref[0]
    whzr1 = whzr_ref[1]
    whzr2 = whzr_ref[2]
    whh0 = whh_ref[0]
    whh1 = whh_ref[1]
    whh2 = whh_ref[2]
    bx = bx_ref[0]
    bhzr = bhzr_ref[0]
    bhh = bhh_ref[0]

    H = jnp.zeros((N, HID), dtype=jnp.float32)
    for t in range(T):
        A = y_ref[0, t, :, :N] * offdiag
        deg = jnp.sum(A, axis=1, keepdims=True)
        dinv = jnp.where(deg > 0,
                         lax.rsqrt(jnp.maximum(deg, 1e-12)),
                         0.0)
        A1 = A * dinv

        def lt_apply(v):
            return -(dinv * _mm_t(A1, v))

        X = y_ref[0, t, :, N:]
        tx1 = lt_apply(X)
        tx2 = 2.0 * lt_apply(tx1) - X
        XC = _mm(X, wx0) + _mm(tx1, wx1) + _mm(tx2, wx2) + bx

        th1 = lt_apply(H)
        th2 = 2.0 * lt_apply(th1) - H
        HCzr = _mm(H, whzr0) + _mm(th1, whzr1) + _mm(th2, whzr2) + bhzr

        Z = jax.nn.sigmoid(XC[:, :HID] + HCzr[:, :HID])
        R = jax.nn.sigmoid(XC[:, HID:2 * HID] + HCzr[:, HID:])

        HR = H * R
        g1 = lt_apply(HR)
        g2 = 2.0 * lt_apply(g1) - HR
        HCh = _mm(HR, whh0) + _mm(g1, whh1) + _mm(g2, whh2) + bhh

        Htil = jnp.tanh(XC[:, 2 * HID:] + HCh)
        H = Z * H + (1.0 - Z) * Htil

    h = jax.nn.relu(_mm(H, wred_ref[...]) + bred_ref[0])
    o = _mm_t(h, wm0_ref[...]) + bm0_ref[...]
    o = _mm(o, wm1_ref[...]) + bm1_ref[...]
    out_ref[0] = o


@jax.jit
def kernel(y, Wxz, bxz, Whz, bhz, Wxr, bxr, Whr, bhr, Wxh, bxh, Whh, bhh,
           Wred, bred, Wm0, bm0, Wm1, bm1):
    B = y.shape[0]
    wx = jnp.concatenate([Wxz, Wxr, Wxh], axis=2)
    bx = jnp.concatenate([bxz, bxr, bxh])[None, :]
    whzr = jnp.concatenate([Whz, Whr], axis=2)
    bhzr = jnp.concatenate([bhz, bhr])[None, :]
    bhh2 = bhh[None, :]
    bred2 = bred[None, :]
    bm02 = bm0[None, :]
    bm12 = bm1[None, :]

    full = lambda shape: pl.BlockSpec(shape, lambda b: (0,) * len(shape))
    out = pl.pallas_call(
        _gru_kernel,
        grid=(B,),
        in_specs=[
            pl.BlockSpec((1, T, N, N + FDIM), lambda b: (b, 0, 0, 0)),
            full((3, FDIM, 3 * HID)),
            full((1, 3 * HID)),
            full((3, HID, 2 * HID)),
            full((1, 2 * HID)),
            full((3, HID, HID)),
            full((1, HID)),
            full((HID, 1)),
            full((1, 1)),
            full((N, 32)),
            full((1, 32)),
            full((32, HID)),
            full((1, HID)),
        ],
        out_specs=pl.BlockSpec((1, 1, HID), lambda b: (b, 0, 0)),
        out_shape=jax.ShapeDtypeStruct((B, 1, HID), jnp.float32),
        compiler_params=pltpu.CompilerParams(
            dimension_semantics=("arbitrary",)),
    )(y, wx, bx, whzr, bhzr, Whh, bhh2, Wred, bred2, Wm0, bm02, Wm1, bm12)
    return out.reshape(B, HID)
